# baseline (speedup 1.0000x reference)
import functools
import jax
import jax.numpy as jnp
import numpy as np
from jax import lax
from jax.experimental import pallas as pl
from jax.experimental.pallas import tpu as pltpu
from jax.experimental.pallas import tpu_sc as plsc

N_HEADS = 4
GAT_HEADS = 6
_D = 512
_N = 10000
_NPAD = 10240
_NPW = 320
_EPAD_EXTRA = 64

_SELU_SCALE = 1.0507009873554805
_SELU_ALPHA = 1.6732632423543772


def _selu(x):
    return _SELU_SCALE * jnp.where(
        x > 0, x, _SELU_ALPHA * (jnp.exp(jnp.minimum(x, 0.0)) - 1.0))


def _layer_norm(h, g, b, eps=1e-5):
    m = jnp.mean(h, axis=-1, keepdims=True)
    v = jnp.var(h, axis=-1, keepdims=True)
    return (h - m) / jnp.sqrt(v + eps) * g + b


def _sget(ref, i):
    return ref[pl.ds(i, 16)][0]


def _lane_gather(v, idx):
    return lax.gather(
        v, idx[:, None],
        dimension_numbers=lax.GatherDimensionNumbers(
            offset_dims=(), collapsed_slice_dims=(0,), start_index_map=(0,)),
        slice_sizes=(1,),
        mode=lax.GatherScatterMode.PROMISE_IN_BOUNDS)


def _unaligned16(buf, sh):
    v0 = buf[pl.ds(0, 16)]
    v1 = buf[pl.ds(8, 16)]
    ii = lax.iota(jnp.int32, 16)
    g0 = _lane_gather(v0, (ii + sh) & 15)
    g1 = _lane_gather(v1, (ii + (sh - 8)) & 15)
    return jnp.where(ii < 16 - sh, g0, g1)


def _hsum_bcast(v):
    ii = lax.iota(jnp.int32, 16)
    for k in (8, 4, 2, 1):
        v = v + _lane_gather(v, ii ^ k)
    return v


def _sc_gat_body(xl_ref, xr_ref, src_ref, off_ref, att_ref, out_ref,
                 offv, attv, idxbuf, xrv, rows, outv, sem):
    cid = lax.axis_index("c")
    sid = lax.axis_index("s")
    wid = sid * 2 + cid
    nlo = wid * _NPW
    nreal = jnp.maximum(jnp.minimum(_N - nlo, _NPW), 0)
    pltpu.sync_copy(att_ref, attv)
    pltpu.sync_copy(off_ref.at[pl.ds(nlo, 336)], offv)

    def node_body(i, carry0):
        d = nlo + i
        e0 = _sget(offv, i)
        e1 = _sget(offv, i + 1)
        deg = e1 - e0
        nch = (deg + 15) // 16
        zz = jnp.zeros((16,), jnp.float32)
        for j in range(32):
            outv[pl.ds(j * 16, 16)] = zz

        def head_body(h, carry1):
            pltpu.sync_copy(xr_ref.at[pl.ds((h * _NPAD + d) * _D, _D)], xrv)

            def chunk_body(c, car):
                p = e0 + c * 16
                ab = (p // 8) * 8
                sh = p - ab
                pltpu.sync_copy(src_ref.at[pl.ds(ab, 24)], idxbuf)
                idx = _unaligned16(idxbuf, sh) + h * _NPAD
                pltpu.async_copy(xl_ref.at[idx], rows, sem).wait()
                cnt = jnp.minimum(16, deg - c * 16)

                def edge_body(e, car2):
                    m, s = car2[0], car2[1]
                    a_acc = car2[2:]
                    pacc = jnp.zeros((16,), jnp.float32)
                    for j in range(32):
                        u = rows[e, pl.ds(j * 16, 16)] + xrv[pl.ds(j * 16, 16)]
                        t = jnp.maximum(u, 0.2 * u)
                        pacc = pacc + t * attv[pl.ds(h * _D + j * 16, 16)]
                    lv = _hsum_bcast(pacc)
                    mn = jnp.maximum(m, lv)
                    c0 = jnp.exp(m - mn)
                    w = jnp.exp(lv - mn)
                    s2 = s * c0 + w
                    new_a = tuple(
                        a_acc[j] * c0 + w * rows[e, pl.ds(j * 16, 16)]
                        for j in range(32))
                    return (mn, s2) + new_a

                return lax.fori_loop(0, cnt, edge_body, car)

            init = (jnp.full((16,), -1e30, jnp.float32),
                    jnp.zeros((16,), jnp.float32)) + tuple(
                        jnp.zeros((16,), jnp.float32) for _ in range(32))
            res = lax.fori_loop(0, nch, chunk_body, init)
            inv = 1.0 / (6.0 * res[1])
            for j in range(32):
                outv[pl.ds(j * 16, 16)] = outv[pl.ds(j * 16, 16)] + res[2 + j] * inv
            return carry1

        lax.fori_loop(0, GAT_HEADS, head_body, 0)
        pltpu.sync_copy(outv, out_ref.at[pl.ds(d * _D, _D)])
        return carry0

    lax.fori_loop(0, nreal, node_body, 0)


@functools.partial(
    pl.kernel,
    out_type=jax.ShapeDtypeStruct((_NPAD * _D,), jnp.float32),
    mesh=plsc.VectorSubcoreMesh(core_axis_name="c", subcore_axis_name="s"),
    scratch_types=[
        pltpu.VMEM((336,), jnp.int32),
        pltpu.VMEM((GAT_HEADS * _D,), jnp.float32),
        pltpu.VMEM((24,), jnp.int32),
        pltpu.VMEM((_D,), jnp.float32),
        pltpu.VMEM((16, _D), jnp.float32),
        pltpu.VMEM((_D,), jnp.float32),
        pltpu.SemaphoreType.DMA,
    ],
)
def _sc_gat(xl_ref, xr_ref, src_ref, off_ref, att_ref, out_ref,
            offv, attv, idxbuf, xrv, rows, outv, sem):
    _sc_gat_body(xl_ref, xr_ref, src_ref, off_ref, att_ref, out_ref,
                 offv, attv, idxbuf, xrv, rows, outv, sem)


def _mm_kernel(a_ref, w_ref, b_ref, o_ref, *, act):
    y = jnp.dot(a_ref[...], w_ref[...], preferred_element_type=jnp.float32)
    y = y + b_ref[...]
    if act == "selu":
        y = _selu(y)
    o_ref[...] = y


def _mm(a, w, b, act="none"):
    m, k = a.shape
    n = w.shape[1]
    bm, bn = 256, min(512, n)
    return pl.pallas_call(
        functools.partial(_mm_kernel, act=act),
        grid=(m // bm, n // bn),
        in_specs=[
            pl.BlockSpec((bm, k), lambda i, j: (i, 0)),
            pl.BlockSpec((k, bn), lambda i, j: (0, j)),
            pl.BlockSpec((1, bn), lambda i, j: (0, j)),
        ],
        out_specs=pl.BlockSpec((bm, bn), lambda i, j: (i, j)),
        out_shape=jax.ShapeDtypeStruct((m, n), jnp.float32),
    )(a, w, b.reshape(1, -1))


def _ew_kernel(x_ref, b_ref, o_ref):
    o_ref[...] = _selu(x_ref[...] + b_ref[...])


def _bias_selu(x, b):
    m, n = x.shape
    return pl.pallas_call(
        _ew_kernel,
        grid=(m // 256,),
        in_specs=[
            pl.BlockSpec((256, n), lambda i: (i, 0)),
            pl.BlockSpec((1, n), lambda i: (0, 0)),
        ],
        out_specs=pl.BlockSpec((256, n), lambda i: (i, 0)),
        out_shape=jax.ShapeDtypeStruct((m, n), jnp.float32),
    )(x, b.reshape(1, -1))


def kernel(x, mesh_feat, conv_feat, edge_index, t_in_w, t_in_b, wq, bq, wk, bk,
           wv, bv, wo, bo, ln1_g, ln1_b, w1, b1, w2, b2, ln2_g, ln2_b,
           t_out_w, t_out_b, lin_w, lin_b, gat_wl, gat_wr, gat_att, gat_b,
           c1_w, c1_b, c2_w, c2_b):
    n = x.shape[0]
    bsz = conv_feat.shape[0]
    d_emb = t_in_w.shape[1]
    head_d = d_emb // N_HEADS
    hidden = lin_w.shape[1]

    mf = mesh_feat.reshape(bsz, -1, mesh_feat.shape[-1])
    h = mf @ t_in_w + t_in_b
    Bb, S, D = h.shape

    def heads(t):
        return t.reshape(Bb, S, N_HEADS, head_d).transpose(0, 2, 1, 3)

    q = heads(h @ wq + bq)
    k = heads(h @ wk + bk)
    v = heads(h @ wv + bv)
    a = jax.nn.softmax(jnp.einsum("bhqd,bhkd->bhqk", q, k) / np.sqrt(head_d), axis=-1)
    o = jnp.einsum("bhqk,bhkd->bhqd", a, v).transpose(0, 2, 1, 3).reshape(Bb, S, D)
    o = o @ wo + bo
    h = _layer_norm(h + o, ln1_g, ln1_b)
    f = jax.nn.relu(h @ w1 + b1) @ w2 + b2
    h = _layer_norm(h + f, ln2_g, ln2_b)
    feats = (h @ t_out_w + t_out_b).reshape(-1, t_out_w.shape[1])

    padrows = _NPAD - n
    feats = jnp.concatenate([x[:, 2:], feats], axis=1)
    featsp = jnp.pad(feats, ((0, padrows), (0, 128 - feats.shape[1])))
    linp = jnp.pad(lin_w, ((0, 128 - lin_w.shape[0]), (0, 0)))
    hid = _mm(featsp, linp, lin_b, act="selu")
    coord2 = jnp.pad(x[:, :2], ((0, padrows), (0, 0)))

    loop = jnp.arange(n, dtype=edge_index.dtype)
    src = jnp.concatenate([edge_index[0], loop])
    dst = jnp.concatenate([edge_index[1], loop])
    e_tot = src.shape[0]
    order = jnp.argsort(dst)
    s_srt = src[order]
    d_srt = dst[order]
    off = jnp.searchsorted(d_srt, jnp.arange(n + 1, dtype=jnp.int32)).astype(jnp.int32)
    off_pad = jnp.full((10304,), e_tot, jnp.int32).at[: n + 1].set(off)
    src_pad = jnp.zeros((e_tot + _EPAD_EXTRA,), jnp.int32).at[:e_tot].set(s_srt)
    att_flat = gat_att.reshape(-1)

    wlp = jnp.pad(gat_wl, ((0, 640 - gat_wl.shape[0]), (0, 0)))
    wrp = jnp.pad(gat_wr, ((0, 640 - gat_wr.shape[0]), (0, 0)))
    zb = jnp.zeros((GAT_HEADS * hidden,), jnp.float32)
    c2wp = jnp.pad(c2_w, ((0, 0), (0, 128 - c2_w.shape[1])))
    c2bp = jnp.pad(c2_b, (0, 128 - c2_b.shape[0]))

    for _ in range(3):
        featp = jnp.concatenate(
            [coord2, hid, jnp.zeros((_NPAD, 126), jnp.float32)], axis=1)
        xl = _mm(featp, wlp, zb).reshape(_NPAD, GAT_HEADS, hidden)
        xr = _mm(featp, wrp, zb).reshape(_NPAD, GAT_HEADS, hidden)
        xl2 = xl.transpose(1, 0, 2).reshape(GAT_HEADS * _NPAD, hidden)
        xr2 = xr.transpose(1, 0, 2).reshape(-1)
        scout = _sc_gat(xl2, xr2, src_pad, off_pad, att_flat)
        hid = _bias_selu(scout.reshape(_NPAD, hidden), gat_b)
        c = _mm(hid, c1_w, c1_b, act="selu")
        coordp = _mm(c, c2wp, c2bp)
        coord2 = coordp[:, :2]
    return coordp[:n, :2]


# speedup vs baseline: 1.0816x; 1.0816x over previous
import functools
import jax
import jax.numpy as jnp
import numpy as np
from jax import lax
from jax.experimental import pallas as pl
from jax.experimental.pallas import tpu as pltpu
from jax.experimental.pallas import tpu_sc as plsc

N_HEADS = 4
GAT_HEADS = 6
_D = 512
_N = 10000
_NPAD = 10240
_NPW = 320
_EPAD_EXTRA = 64

_SELU_SCALE = 1.0507009873554805
_SELU_ALPHA = 1.6732632423543772


def _selu(x):
    return _SELU_SCALE * jnp.where(
        x > 0, x, _SELU_ALPHA * (jnp.exp(jnp.minimum(x, 0.0)) - 1.0))


def _layer_norm(h, g, b, eps=1e-5):
    m = jnp.mean(h, axis=-1, keepdims=True)
    v = jnp.var(h, axis=-1, keepdims=True)
    return (h - m) / jnp.sqrt(v + eps) * g + b


def _sget(ref, i):
    return ref[pl.ds(i, 16)][0]


def _lane_gather(v, idx):
    return lax.gather(
        v, idx[:, None],
        dimension_numbers=lax.GatherDimensionNumbers(
            offset_dims=(), collapsed_slice_dims=(0,), start_index_map=(0,)),
        slice_sizes=(1,),
        mode=lax.GatherScatterMode.PROMISE_IN_BOUNDS)


def _unaligned16(buf, base, sh):
    v0 = buf[pl.ds(base, 16)]
    v1 = buf[pl.ds(base + 8, 16)]
    ii = lax.iota(jnp.int32, 16)
    g0 = _lane_gather(v0, (ii + sh) & 15)
    g1 = _lane_gather(v1, (ii + (sh - 8)) & 15)
    return jnp.where(ii < 16 - sh, g0, g1)


def _hsum_bcast(v):
    ii = lax.iota(jnp.int32, 16)
    for k in (8, 4, 2, 1):
        v = v + _lane_gather(v, ii ^ k)
    return v


def _sc_gat_body(xl_ref, xr_ref, src_ref, off_ref, att_ref, out_ref,
                 offv, attv, idxbuf, idxv, xrv, rows, outv, sem):
    cid = lax.axis_index("c")
    sid = lax.axis_index("s")
    wid = sid * 2 + cid
    nlo = wid * _NPW
    nreal = jnp.maximum(jnp.minimum(_N - nlo, _NPW), 0)
    pltpu.sync_copy(att_ref, attv)
    pltpu.sync_copy(off_ref.at[pl.ds(nlo, 336)], offv)

    def node_body(i, carry0):
        d = nlo + i
        e0 = _sget(offv, i)
        e1 = _sget(offv, i + 1)
        deg = e1 - e0
        nch = (deg + 31) // 32
        zz = jnp.zeros((16,), jnp.float32)
        for j in range(32):
            outv[pl.ds(j * 16, 16)] = zz
        pltpu.sync_copy(
            xr_ref.at[pl.ds(d * (GAT_HEADS * _D), GAT_HEADS * _D)], xrv)

        def head_body(h, carry1):

            def chunk_body(c, car):
                p = e0 + c * 32
                ab = (p // 8) * 8
                sh = p - ab
                pltpu.sync_copy(src_ref.at[pl.ds(ab, 48)], idxbuf)
                hb = h * _NPAD
                idxv[pl.ds(0, 16)] = _unaligned16(idxbuf, 0, sh) + hb
                idxv[pl.ds(16, 16)] = _unaligned16(idxbuf, 16, sh) + hb
                pltpu.async_copy(xl_ref.at[idxv], rows, sem).wait()
                cnt = jnp.minimum(32, deg - c * 32)

                def edge_body(e, car2):
                    m, s = car2[0], car2[1]
                    a_acc = car2[2:]
                    pacc = jnp.zeros((16,), jnp.float32)
                    for j in range(32):
                        u = rows[e, pl.ds(j * 16, 16)] + xrv[pl.ds(h * _D + j * 16, 16)]
                        t = jnp.maximum(u, 0.2 * u)
                        pacc = pacc + t * attv[pl.ds(h * _D + j * 16, 16)]
                    lv = _hsum_bcast(pacc)
                    mn = jnp.maximum(m, lv)
                    c0 = jnp.exp(m - mn)
                    w = jnp.exp(lv - mn)
                    s2 = s * c0 + w
                    new_a = tuple(
                        a_acc[j] * c0 + w * rows[e, pl.ds(j * 16, 16)]
                        for j in range(32))
                    return (mn, s2) + new_a

                return lax.fori_loop(0, cnt, edge_body, car)

            init = (jnp.full((16,), -1e30, jnp.float32),
                    jnp.zeros((16,), jnp.float32)) + tuple(
                        jnp.zeros((16,), jnp.float32) for _ in range(32))
            res = lax.fori_loop(0, nch, chunk_body, init)
            inv = 1.0 / (6.0 * res[1])
            for j in range(32):
                outv[pl.ds(j * 16, 16)] = outv[pl.ds(j * 16, 16)] + res[2 + j] * inv
            return carry1

        lax.fori_loop(0, GAT_HEADS, head_body, 0)
        pltpu.sync_copy(outv, out_ref.at[pl.ds(d * _D, _D)])
        return carry0

    lax.fori_loop(0, nreal, node_body, 0)


@functools.partial(
    pl.kernel,
    out_type=jax.ShapeDtypeStruct((_NPAD * _D,), jnp.float32),
    mesh=plsc.VectorSubcoreMesh(core_axis_name="c", subcore_axis_name="s"),
    scratch_types=[
        pltpu.VMEM((336,), jnp.int32),
        pltpu.VMEM((GAT_HEADS * _D,), jnp.float32),
        pltpu.VMEM((48,), jnp.int32),
        pltpu.VMEM((32,), jnp.int32),
        pltpu.VMEM((GAT_HEADS * _D,), jnp.float32),
        pltpu.VMEM((32, _D), jnp.float32),
        pltpu.VMEM((_D,), jnp.float32),
        pltpu.SemaphoreType.DMA,
    ],
)
def _sc_gat(xl_ref, xr_ref, src_ref, off_ref, att_ref, out_ref,
            offv, attv, idxbuf, idxv, xrv, rows, outv, sem):
    _sc_gat_body(xl_ref, xr_ref, src_ref, off_ref, att_ref, out_ref,
                 offv, attv, idxbuf, idxv, xrv, rows, outv, sem)


def _mm_kernel(a_ref, w_ref, b_ref, o_ref, *, act):
    y = jnp.dot(a_ref[...], w_ref[...], preferred_element_type=jnp.float32)
    y = y + b_ref[...]
    if act == "selu":
        y = _selu(y)
    o_ref[...] = y


def _mm(a, w, b, act="none"):
    m, k = a.shape
    n = w.shape[1]
    bm, bn = 256, min(512, n)
    return pl.pallas_call(
        functools.partial(_mm_kernel, act=act),
        grid=(m // bm, n // bn),
        in_specs=[
            pl.BlockSpec((bm, k), lambda i, j: (i, 0)),
            pl.BlockSpec((k, bn), lambda i, j: (0, j)),
            pl.BlockSpec((1, bn), lambda i, j: (0, j)),
        ],
        out_specs=pl.BlockSpec((bm, bn), lambda i, j: (i, j)),
        out_shape=jax.ShapeDtypeStruct((m, n), jnp.float32),
    )(a, w, b.reshape(1, -1))


def _ew_kernel(x_ref, b_ref, o_ref):
    o_ref[...] = _selu(x_ref[...] + b_ref[...])


def _bias_selu(x, b):
    m, n = x.shape
    return pl.pallas_call(
        _ew_kernel,
        grid=(m // 256,),
        in_specs=[
            pl.BlockSpec((256, n), lambda i: (i, 0)),
            pl.BlockSpec((1, n), lambda i: (0, 0)),
        ],
        out_specs=pl.BlockSpec((256, n), lambda i: (i, 0)),
        out_shape=jax.ShapeDtypeStruct((m, n), jnp.float32),
    )(x, b.reshape(1, -1))


def kernel(x, mesh_feat, conv_feat, edge_index, t_in_w, t_in_b, wq, bq, wk, bk,
           wv, bv, wo, bo, ln1_g, ln1_b, w1, b1, w2, b2, ln2_g, ln2_b,
           t_out_w, t_out_b, lin_w, lin_b, gat_wl, gat_wr, gat_att, gat_b,
           c1_w, c1_b, c2_w, c2_b):
    n = x.shape[0]
    bsz = conv_feat.shape[0]
    d_emb = t_in_w.shape[1]
    head_d = d_emb // N_HEADS
    hidden = lin_w.shape[1]

    mf = mesh_feat.reshape(bsz, -1, mesh_feat.shape[-1])
    h = mf @ t_in_w + t_in_b
    Bb, S, D = h.shape

    def heads(t):
        return t.reshape(Bb, S, N_HEADS, head_d).transpose(0, 2, 1, 3)

    q = heads(h @ wq + bq)
    k = heads(h @ wk + bk)
    v = heads(h @ wv + bv)
    a = jax.nn.softmax(jnp.einsum("bhqd,bhkd->bhqk", q, k) / np.sqrt(head_d), axis=-1)
    o = jnp.einsum("bhqk,bhkd->bhqd", a, v).transpose(0, 2, 1, 3).reshape(Bb, S, D)
    o = o @ wo + bo
    h = _layer_norm(h + o, ln1_g, ln1_b)
    f = jax.nn.relu(h @ w1 + b1) @ w2 + b2
    h = _layer_norm(h + f, ln2_g, ln2_b)
    feats = (h @ t_out_w + t_out_b).reshape(-1, t_out_w.shape[1])

    padrows = _NPAD - n
    feats = jnp.concatenate([x[:, 2:], feats], axis=1)
    featsp = jnp.pad(feats, ((0, padrows), (0, 128 - feats.shape[1])))
    linp = jnp.pad(lin_w, ((0, 128 - lin_w.shape[0]), (0, 0)))
    hid = _mm(featsp, linp, lin_b, act="selu")
    coord2 = jnp.pad(x[:, :2], ((0, padrows), (0, 0)))

    loop = jnp.arange(n, dtype=edge_index.dtype)
    src = jnp.concatenate([edge_index[0], loop])
    dst = jnp.concatenate([edge_index[1], loop])
    e_tot = src.shape[0]
    order = jnp.argsort(dst)
    s_srt = src[order]
    d_srt = dst[order]
    off = jnp.searchsorted(d_srt, jnp.arange(n + 1, dtype=jnp.int32)).astype(jnp.int32)
    off_pad = jnp.full((10304,), e_tot, jnp.int32).at[: n + 1].set(off)
    src_pad = jnp.zeros((e_tot + _EPAD_EXTRA,), jnp.int32).at[:e_tot].set(s_srt)
    att_flat = gat_att.reshape(-1)

    wlp = jnp.pad(gat_wl, ((0, 640 - gat_wl.shape[0]), (0, 0)))
    wrp = jnp.pad(gat_wr, ((0, 640 - gat_wr.shape[0]), (0, 0)))
    zb = jnp.zeros((GAT_HEADS * hidden,), jnp.float32)
    c2wp = jnp.pad(c2_w, ((0, 0), (0, 128 - c2_w.shape[1])))
    c2bp = jnp.pad(c2_b, (0, 128 - c2_b.shape[0]))

    for _ in range(3):
        featp = jnp.concatenate(
            [coord2, hid, jnp.zeros((_NPAD, 126), jnp.float32)], axis=1)
        xl = _mm(featp, wlp, zb).reshape(_NPAD, GAT_HEADS, hidden)
        xr = _mm(featp, wrp, zb).reshape(_NPAD, GAT_HEADS, hidden)
        xl2 = xl.transpose(1, 0, 2).reshape(GAT_HEADS * _NPAD, hidden)
        xr2 = xr.reshape(-1)
        scout = _sc_gat(xl2, xr2, src_pad, off_pad, att_flat)
        hid = _bias_selu(scout.reshape(_NPAD, hidden), gat_b)
        c = _mm(hid, c1_w, c1_b, act="selu")
        coordp = _mm(c, c2wp, c2bp)
        coord2 = coordp[:, :2]
    return coordp[:n, :2]
